# baseline (speedup 1.0000x reference)
import functools

import jax
import jax.numpy as jnp
from jax import lax
from jax.experimental import pallas as pl
from jax.experimental.pallas import tpu as pltpu
from jax.experimental.pallas import tpu_sc as plsc

_E = 320000
_NW = 32
_EPW = _E // _NW
_CHUNKS = _EPW // 16
_SPLIT = 320
_UNROLL = 5
_ALPHA = 0.01


@functools.lru_cache(maxsize=None)
def _get_sc_edge_reduce():
    mesh = plsc.VectorSubcoreMesh(core_axis_name="c", subcore_axis_name="s")

    @functools.partial(
        pl.kernel,
        mesh=mesh,
        out_type=jax.ShapeDtypeStruct((_NW * 80,), jnp.float32),
        scratch_types=[
            pltpu.VMEM((_EPW,), jnp.int32),
            pltpu.VMEM((4 * _EPW,), jnp.float32),
            pltpu.VMEM((80,), jnp.float32),
            pltpu.SemaphoreType.DMA,
            pltpu.SemaphoreType.DMA,
            pltpu.SemaphoreType.DMA,
        ],
    )
    def _sc_edge_reduce(recv_hbm, ecols_hbm, out_hbm, r_v, e_v, o_v,
                        sem_r, sem_a, sem_b):
        wid = lax.axis_index("c") * 16 + lax.axis_index("s")
        n1 = _SPLIT * 16
        n2 = _EPW - n1
        cp_r = pltpu.async_copy(recv_hbm.at[pl.ds(wid * _EPW, _EPW)], r_v, sem_r)
        cps_a = []
        cps_b = []
        for f in range(4):
            src = f * _E + wid * _EPW
            cps_a.append(pltpu.async_copy(
                ecols_hbm.at[pl.ds(src, n1)],
                e_v.at[pl.ds(f * _EPW, n1)], sem_a))
            cps_b.append(pltpu.async_copy(
                ecols_hbm.at[pl.ds(src + n1, n2)],
                e_v.at[pl.ds(f * _EPW + n1, n2)], sem_b))
        cp_r.wait()
        for cp in cps_a:
            cp.wait()

        zero = jnp.zeros((16,), jnp.float32)
        ones = jnp.full((16,), 1.0, jnp.float32)
        zi = jnp.zeros((16,), jnp.int32)

        def body(i, carry):
            a0, a1, a2, a3, cn = carry
            for u in range(_UNROLL):
                base = (i * _UNROLL + u) * 16
                r16 = r_v[pl.ds(base, 16)]
                m = r16 == zi
                cn = cn + jnp.where(m, ones, zero)
                accs = []
                for f, a in enumerate((a0, a1, a2, a3)):
                    ev = e_v[pl.ds(f * _EPW + base, 16)]
                    accs.append(a + jnp.where(m, ev, zero))
                a0, a1, a2, a3 = accs
            return (a0, a1, a2, a3, cn)

        init = (zero, zero, zero, zero, zero)
        carry = lax.fori_loop(0, _SPLIT // _UNROLL, body, init)
        for cp in cps_b:
            cp.wait()
        a0, a1, a2, a3, cn = lax.fori_loop(
            _SPLIT // _UNROLL, _CHUNKS // _UNROLL, body, carry)
        o_v[pl.ds(0, 16)] = a0
        o_v[pl.ds(16, 16)] = a1
        o_v[pl.ds(32, 16)] = a2
        o_v[pl.ds(48, 16)] = a3
        o_v[pl.ds(64, 16)] = cn
        pltpu.sync_copy(o_v, out_hbm.at[pl.ds(wid * 80, 80)])

    return _sc_edge_reduce


def _dotT(a, b):
    return lax.dot_general(a, b, (((1,), (1,)), ((), ())),
                           precision=lax.Precision.HIGHEST,
                           preferred_element_type=jnp.float32)


def _mm(a, b):
    return lax.dot_general(a, b, (((1,), (0,)), ((), ())),
                           precision=lax.Precision.HIGHEST,
                           preferred_element_type=jnp.float32)


def _lrelu(x):
    return jnp.where(x >= 0, x, _ALPHA * x)


def _dense_body(part_ref, n0_ref, wne_ref, bne_ref, wee_ref, bee_ref,
                wm_ref, bm_ref, wu_ref, bu_ref,
                wo1_ref, bo1_ref, wo2_ref, bo2_ref, wo3_ref, bo3_ref,
                wo3t_ref, wne36_ref, h_ref, g_ref):
    part = part_ref[...]
    s0 = jnp.sum(part[:, 0:16])
    s1 = jnp.sum(part[:, 16:32])
    s2 = jnp.sum(part[:, 32:48])
    s3 = jnp.sum(part[:, 48:64])
    d0 = jnp.sum(part[:, 64:80])
    wee = wee_ref[...]
    ee_agg = (s0 * wee[0:1, :] + s1 * wee[1:2, :] + s2 * wee[2:3, :]
              + s3 * wee[3:4, :] + d0 * bee_ref[...])

    ne = _mm(n0_ref[...], wne_ref[...]) + bne_ref[...]
    z0 = ne
    for i in range(3):
        agg = _mm(ee_agg, wm_ref[i]) + d0 * bm_ref[i]
        comb = jnp.concatenate([ne, agg], axis=1)
        ne = _lrelu(_mm(comb, wu_ref[i]) + bu_ref[i])

    a1 = _lrelu(_mm(ne, wo1_ref[...]) + bo1_ref[...])
    a2 = _lrelu(_mm(a1, wo2_ref[...]) + bo2_ref[...])
    h_ref[...] = _mm(a2, wo3_ref[...]) + bo3_ref[...]

    t1 = _mm(z0, wo1_ref[...]) + bo1_ref[...]
    u1 = jnp.where(t1 >= 0, 1.0, _ALPHA)
    t2 = _mm(_lrelu(t1), wo2_ref[...]) + bo2_ref[...]
    u2 = jnp.where(t2 >= 0, 1.0, _ALPHA)
    g = wo3t_ref[...] * u2
    g = _dotT(g, wo2_ref[...]) * u1
    g = _dotT(g, wo1_ref[...])
    g_ref[...] = _dotT(g, wne36_ref[...])


_dense_call = pl.pallas_call(
    _dense_body,
    out_shape=[jax.ShapeDtypeStruct((1, 1), jnp.float32),
               jax.ShapeDtypeStruct((1, 3), jnp.float32)],
)


@jax.jit
def kernel(nodes, edges, receivers, W_ne, b_ne, W_ee, b_ee, W_m, b_m,
           W_u, b_u, W_a, b_a, W_o1, b_o1, W_o2, b_o2, W_o3, b_o3):
    del W_a, b_a
    ecols = edges.T.reshape(-1)
    part = _get_sc_edge_reduce()(receivers, ecols)
    part = part.reshape(_NW, 80)
    h, grad = _dense_call(
        part, nodes[0:1], W_ne, b_ne.reshape(1, -1), W_ee,
        b_ee.reshape(1, -1), W_m, b_m.reshape(3, 1, -1), W_u,
        b_u.reshape(3, 1, -1), W_o1, b_o1.reshape(1, -1), W_o2,
        b_o2.reshape(1, -1), W_o3, b_o3.reshape(1, -1),
        W_o3.T, W_ne[3:6])
    return (h[0, 0], grad[0])


# speedup vs baseline: 1.0600x; 1.0600x over previous
import functools

import jax
import jax.numpy as jnp
from jax import lax
from jax.experimental import pallas as pl
from jax.experimental.pallas import tpu as pltpu
from jax.experimental.pallas import tpu_sc as plsc

_E = 320000
_NW = 32
_EPW = _E // _NW
_CHUNKS = _EPW // 16
_SPLIT = 320
_UNROLL = 5
_ALPHA = 0.01


@functools.lru_cache(maxsize=None)
def _get_sc_edge_reduce():
    mesh = plsc.VectorSubcoreMesh(core_axis_name="c", subcore_axis_name="s")

    @functools.partial(
        pl.kernel,
        mesh=mesh,
        out_type=jax.ShapeDtypeStruct((_NW, 80), jnp.float32),
        scratch_types=[
            pltpu.VMEM((_EPW,), jnp.int32),
            pltpu.VMEM((4 * _EPW,), jnp.float32),
            pltpu.VMEM((80,), jnp.float32),
            pltpu.SemaphoreType.DMA,
            pltpu.SemaphoreType.DMA,
            pltpu.SemaphoreType.DMA,
        ],
    )
    def _sc_edge_reduce(recv_hbm, ecols_hbm, out_hbm, r_v, e_v, o_v,
                        sem_r, sem_a, sem_b):
        wid = lax.axis_index("c") * 16 + lax.axis_index("s")
        n1 = _SPLIT * 16
        n2 = _EPW - n1
        cp_r = pltpu.async_copy(recv_hbm.at[pl.ds(wid * _EPW, _EPW)], r_v, sem_r)
        cps_a = []
        cps_b = []
        for f in range(4):
            src = f * _E + wid * _EPW
            cps_a.append(pltpu.async_copy(
                ecols_hbm.at[pl.ds(src, n1)],
                e_v.at[pl.ds(f * _EPW, n1)], sem_a))
            cps_b.append(pltpu.async_copy(
                ecols_hbm.at[pl.ds(src + n1, n2)],
                e_v.at[pl.ds(f * _EPW + n1, n2)], sem_b))
        cp_r.wait()
        for cp in cps_a:
            cp.wait()

        zero = jnp.zeros((16,), jnp.float32)
        ones = jnp.full((16,), 1.0, jnp.float32)
        zi = jnp.zeros((16,), jnp.int32)

        def body(i, carry):
            a0, a1, a2, a3, cn = carry
            for u in range(_UNROLL):
                base = (i * _UNROLL + u) * 16
                r16 = r_v[pl.ds(base, 16)]
                m = r16 == zi
                cn = cn + jnp.where(m, ones, zero)
                accs = []
                for f, a in enumerate((a0, a1, a2, a3)):
                    ev = e_v[pl.ds(f * _EPW + base, 16)]
                    accs.append(a + jnp.where(m, ev, zero))
                a0, a1, a2, a3 = accs
            return (a0, a1, a2, a3, cn)

        init = (zero, zero, zero, zero, zero)
        carry = lax.fori_loop(0, _SPLIT // _UNROLL, body, init)
        for cp in cps_b:
            cp.wait()
        a0, a1, a2, a3, cn = lax.fori_loop(
            _SPLIT // _UNROLL, _CHUNKS // _UNROLL, body, carry)
        o_v[pl.ds(0, 16)] = a0
        o_v[pl.ds(16, 16)] = a1
        o_v[pl.ds(32, 16)] = a2
        o_v[pl.ds(48, 16)] = a3
        o_v[pl.ds(64, 16)] = cn
        pltpu.sync_copy(o_v, out_hbm.at[wid])

    return _sc_edge_reduce


def _dotT(a, b):
    return lax.dot_general(a, b, (((1,), (1,)), ((), ())),
                           precision=lax.Precision.HIGHEST,
                           preferred_element_type=jnp.float32)


def _mm(a, b):
    return lax.dot_general(a, b, (((1,), (0,)), ((), ())),
                           precision=lax.Precision.HIGHEST,
                           preferred_element_type=jnp.float32)


def _lrelu(x):
    return jnp.where(x >= 0, x, _ALPHA * x)


def _mm_b16(a, b):
    return lax.dot_general(a.astype(jnp.bfloat16), b.astype(jnp.bfloat16),
                           (((1,), (0,)), ((), ())),
                           preferred_element_type=jnp.float32)


def _dotT_b16(a, b):
    return lax.dot_general(a.astype(jnp.bfloat16), b.astype(jnp.bfloat16),
                           (((1,), (1,)), ((), ())),
                           preferred_element_type=jnp.float32)


def _dense_body(part_ref, n0_ref, wne_ref, bne_ref, wee_ref, bee_ref,
                wm_ref, bm_ref, wu_ref, bu_ref,
                wo1_ref, bo1_ref, wo2_ref, bo2_ref, wo3_ref, bo3_ref,
                wo3t_ref, wne36_ref, h_ref, g_ref):
    part = part_ref[...]
    s0 = jnp.sum(part[:, 0:16])
    s1 = jnp.sum(part[:, 16:32])
    s2 = jnp.sum(part[:, 32:48])
    s3 = jnp.sum(part[:, 48:64])
    d0 = jnp.sum(part[:, 64:80])
    def _b16(x):
        return x.astype(jnp.bfloat16).astype(jnp.float32)

    wee = _b16(wee_ref[...])
    ee_agg = (s0 * wee[0:1, :] + s1 * wee[1:2, :] + s2 * wee[2:3, :]
              + s3 * wee[3:4, :] + d0 * bee_ref[...])

    ne = _mm_b16(n0_ref[...], wne_ref[...]) + bne_ref[...]
    for i in range(3):
        agg = _mm(ee_agg, _b16(wm_ref[i])) + d0 * bm_ref[i]
        comb = jnp.concatenate([ne, agg], axis=1)
        ne = _lrelu(_mm_b16(comb, wu_ref[i]) + bu_ref[i])

    a1 = _lrelu(_mm_b16(ne, wo1_ref[...]) + bo1_ref[...])
    a2 = _lrelu(_mm_b16(a1, wo2_ref[...]) + bo2_ref[...])
    h_ref[...] = _mm_b16(a2, wo3_ref[...]) + bo3_ref[...]

    z0b = _mm_b16(n0_ref[...], wne_ref[...]) + bne_ref[...]
    t1b = _mm_b16(z0b, wo1_ref[...]) + bo1_ref[...]
    u1 = jnp.where(t1b >= 0, 1.0, _ALPHA)
    t2b = _mm_b16(_lrelu(t1b), wo2_ref[...]) + bo2_ref[...]
    u2 = jnp.where(t2b >= 0, 1.0, _ALPHA)
    g = wo3t_ref[...] * u2
    g = _dotT_b16(g, wo2_ref[...]) * u1
    g = _dotT_b16(g, wo1_ref[...])
    g_ref[...] = _dotT_b16(g, wne36_ref[...])


_dense_call = pl.pallas_call(
    _dense_body,
    out_shape=[jax.ShapeDtypeStruct((1, 1), jnp.float32),
               jax.ShapeDtypeStruct((1, 3), jnp.float32)],
)


@jax.jit
def kernel(nodes, edges, receivers, W_ne, b_ne, W_ee, b_ee, W_m, b_m,
           W_u, b_u, W_a, b_a, W_o1, b_o1, W_o2, b_o2, W_o3, b_o3):
    del W_a, b_a
    ecols = edges.T.reshape(-1)
    part = _get_sc_edge_reduce()(receivers, ecols)
    h, grad = _dense_call(
        part, nodes[0:1], W_ne, b_ne.reshape(1, -1), W_ee,
        b_ee.reshape(1, -1), W_m, b_m.reshape(3, 1, -1), W_u,
        b_u.reshape(3, 1, -1), W_o1, b_o1.reshape(1, -1), W_o2,
        b_o2.reshape(1, -1), W_o3, b_o3.reshape(1, -1),
        W_o3.T, W_ne[3:6])
    return (h[0, 0], grad[0])
